# baseline (speedup 1.0000x reference)
import functools

import jax
import jax.numpy as jnp
from jax import lax
from jax.experimental import pallas as pl
from jax.experimental.pallas import tpu as pltpu
from jax.experimental.pallas import tpu_sc as plsc

RES = 16
D = 64
B = 32
N = 8192
TABLE = RES ** 3

NC = 2
NS = 16
DG = 8
NB_SC = 16

LANES = 16
PAD = 16
HA = 4096
WB = N + 1 - HA


@functools.partial(
    pl.pallas_call,
    out_shape=jax.ShapeDtypeStruct((B * N,), jnp.int32),
    grid=(B // 8,),
    in_specs=[
        pl.BlockSpec((8, N), lambda q: (q, 0)),
        pl.BlockSpec((8, N), lambda q: (q, 0)),
        pl.BlockSpec((8, N), lambda q: (q, 0)),
    ],
    out_specs=pl.BlockSpec((8 * N,), lambda q: (q,)),
)
def _pos_kernel(x_ref, y_ref, z_ref, o_ref):
    xi = jnp.round(x_ref[...] * 15.0).astype(jnp.int32)
    yi = jnp.round(y_ref[...] * 15.0).astype(jnp.int32)
    zi = jnp.round(z_ref[...] * 15.0).astype(jnp.int32)
    o_ref[...] = (xi * 256 + yi * 16 + zi).reshape(8 * N)


def _sc_body(pos_hbm, op_hbm, t_hbm, out_hbm,
             idxv, tvm, bufA, bufB, opvm,
             sem_tab, sem_idx, sem_wA, sem_wB):
    c = lax.axis_index("c")
    s = lax.axis_index("s")
    g = s % DG
    sg = s // DG
    d0 = pl.multiple_of(g * DG, DG)

    iota = lax.iota(jnp.int32, LANES)

    tab_cp = pltpu.async_copy(
        t_hbm.at[pl.ds(pl.multiple_of(g * (DG * TABLE), 128), DG * TABLE)],
        tvm, sem_tab)
    pltpu.sync_copy(op_hbm, opvm)

    idxv[pl.ds(0, PAD)] = jnp.zeros((PAD,), jnp.int32)
    idxv[pl.ds(PAD + N, LANES)] = jnp.zeros((LANES,), jnp.int32)

    idx_cp = pltpu.async_copy(
        pos_hbm.at[pl.ds(pl.multiple_of((c * NB_SC + sg * DG) * N, N), N)],
        idxv.at[pl.ds(PAD, N)], sem_idx)
    tab_cp.wait()

    pend_A = pend_B = None
    for k in range(DG):
        b = c * NB_SC + sg * DG + k
        idx_cp.wait()

        if pend_A is not None:
            pend_A.wait()

        @plsc.parallel_loop(0, HA // LANES, unroll=8)
        def _stepA(j):
            iv = plsc.load_gather(idxv, [iota + (j * LANES + (PAD - 1))])
            off = pl.multiple_of(j * LANES, LANES)
            for dd in range(DG):
                gv = plsc.load_gather(tvm, [iv + (dd * TABLE)])
                bufA[dd, pl.ds(off, LANES)] = gv

        opvec = opvm[pl.ds(d0, LANES)]
        plsc.store_scatter(bufA, [iota, jnp.zeros((LANES,), jnp.int32)],
                           opvec, mask=iota < DG)
        pend_A = pltpu.async_copy(
            bufA, out_hbm.at[b, pl.ds(d0, DG), pl.ds(0, HA)], sem_wA)

        if pend_B is not None:
            pend_B.wait()

        @plsc.parallel_loop(0, (WB - 1) // LANES, unroll=8)
        def _stepB(j):
            iv = plsc.load_gather(idxv,
                                  [iota + (j * LANES + (PAD - 1 + HA))])
            off = pl.multiple_of(j * LANES, LANES)
            for dd in range(DG):
                gv = plsc.load_gather(tvm, [iv + (dd * TABLE)])
                bufB[dd, pl.ds(off, LANES)] = gv

        ivt = plsc.load_gather(idxv, [iota + (PAD - 1 + N)])
        for dd in range(DG):
            gv = plsc.load_gather(tvm, [ivt + (dd * TABLE)])
            plsc.store_scatter(bufB, [jnp.full((LANES,), dd, jnp.int32),
                                      jnp.full((LANES,), WB - 1, jnp.int32)],
                               gv, mask=iota < 1)
        if k + 1 < DG:
            nxt = pl.multiple_of((c * NB_SC + sg * DG + (k + 1)) * N, N)
            idx_cp = pltpu.async_copy(pos_hbm.at[pl.ds(nxt, N)],
                                      idxv.at[pl.ds(PAD, N)], sem_idx)
        pend_B = pltpu.async_copy(
            bufB, out_hbm.at[b, pl.ds(d0, DG), pl.ds(HA, WB)], sem_wB)

    pend_A.wait()
    pend_B.wait()


@functools.partial(
    pl.kernel,
    mesh=plsc.VectorSubcoreMesh(core_axis_name="c", subcore_axis_name="s"),
    compiler_params=pltpu.CompilerParams(use_tc_tiling_on_sc=True,
                                         needs_layout_passes=False),
    out_type=jax.ShapeDtypeStruct((B, D, N + 1), jnp.float32),
    scratch_types=[
        pltpu.VMEM((PAD + N + LANES,), jnp.int32),
        pltpu.VMEM((DG * TABLE,), jnp.float32),
        pltpu.VMEM((DG, HA), jnp.float32),
        pltpu.VMEM((DG, WB), jnp.float32),
        pltpu.VMEM((D + LANES,), jnp.float32),
        pltpu.SemaphoreType.DMA,
        pltpu.SemaphoreType.DMA,
        pltpu.SemaphoreType.DMA,
        pltpu.SemaphoreType.DMA,
    ],
)
def _sc_kernel(pos_hbm, op_hbm, t_hbm, out_hbm,
               idxv, tvm, bufA, bufB, opvm,
               sem_tab, sem_idx, sem_wA, sem_wB):
    _sc_body(pos_hbm, op_hbm, t_hbm, out_hbm,
             idxv, tvm, bufA, bufB, opvm,
             sem_tab, sem_idx, sem_wA, sem_wB)


def kernel(x, y, z, outputPos, posEmbeddingList):
    pos = _pos_kernel(x, y, z)
    tflat = posEmbeddingList.T.reshape(-1)
    op_pad = jnp.pad(outputPos.reshape(-1), (0, LANES))
    out_t = _sc_kernel(pos, op_pad, tflat)
    return out_t.transpose(0, 2, 1)


# speedup vs baseline: 1.1800x; 1.1800x over previous
import functools

import jax
import jax.numpy as jnp
from jax import lax
from jax.experimental import pallas as pl
from jax.experimental.pallas import tpu as pltpu
from jax.experimental.pallas import tpu_sc as plsc

RES = 16
D = 64
B = 32
N = 8192
TABLE = RES ** 3

NC = 2
NS = 16
DG = 8
NB_SC = 16

LANES = 16
PAD = 16
HA = 4096
WB = N + 1 - HA


@functools.partial(
    pl.pallas_call,
    out_shape=jax.ShapeDtypeStruct((B * N,), jnp.int32),
    grid=(B // 8,),
    in_specs=[
        pl.BlockSpec((8, N), lambda q: (q, 0)),
        pl.BlockSpec((8, N), lambda q: (q, 0)),
        pl.BlockSpec((8, N), lambda q: (q, 0)),
    ],
    out_specs=pl.BlockSpec((8 * N,), lambda q: (q,)),
)
def _pos_kernel(x_ref, y_ref, z_ref, o_ref):
    xi = jnp.round(x_ref[...] * 15.0).astype(jnp.int32)
    yi = jnp.round(y_ref[...] * 15.0).astype(jnp.int32)
    zi = jnp.round(z_ref[...] * 15.0).astype(jnp.int32)
    o_ref[...] = (xi * 256 + yi * 16 + zi).reshape(8 * N)


def _sc_body(pos_hbm, op_hbm, t_hbm, out_hbm,
             idxv0, idxv1, tvm, bufA, bufB, opvm,
             sem_tab, sem_idx0, sem_idx1, sem_wA, sem_wB):
    c = lax.axis_index("c")
    s = lax.axis_index("s")
    g = s % DG
    sg = s // DG
    d0 = pl.multiple_of(g * DG, DG)

    iota = lax.iota(jnp.int32, LANES)

    tab_cp = pltpu.async_copy(
        t_hbm.at[pl.ds(pl.multiple_of(g * (DG * TABLE), 128), DG * TABLE)],
        tvm, sem_tab)
    pltpu.sync_copy(op_hbm, opvm)

    ibufs = (idxv0, idxv1)
    isems = (sem_idx0, sem_idx1)
    for ib in ibufs:
        ib[pl.ds(0, PAD)] = jnp.zeros((PAD,), jnp.int32)
        ib[pl.ds(PAD + N, LANES)] = jnp.zeros((LANES,), jnp.int32)

    icps = [pltpu.async_copy(
        pos_hbm.at[pl.ds(pl.multiple_of((c * NB_SC + sg * DG) * N, N), N)],
        idxv0.at[pl.ds(PAD, N)], sem_idx0), None]
    tab_cp.wait()

    pend_A = pend_B = None
    for k in range(DG):
        b = c * NB_SC + sg * DG + k
        idxv = ibufs[k % 2]
        icps[k % 2].wait()
        if k + 1 < DG:
            nxt = pl.multiple_of((c * NB_SC + sg * DG + (k + 1)) * N, N)
            icps[(k + 1) % 2] = pltpu.async_copy(
                pos_hbm.at[pl.ds(nxt, N)],
                ibufs[(k + 1) % 2].at[pl.ds(PAD, N)], isems[(k + 1) % 2])

        if pend_A is not None:
            pend_A.wait()

        @plsc.parallel_loop(0, HA // LANES, unroll=8)
        def _stepA(j):
            iv = plsc.load_gather(idxv, [iota + (j * LANES + (PAD - 1))])
            off = pl.multiple_of(j * LANES, LANES)
            for dd in range(DG):
                gv = plsc.load_gather(tvm, [iv + (dd * TABLE)])
                bufA[dd, pl.ds(off, LANES)] = gv

        opvec = opvm[pl.ds(d0, LANES)]
        plsc.store_scatter(bufA, [iota, jnp.zeros((LANES,), jnp.int32)],
                           opvec, mask=iota < DG)
        pend_A = pltpu.async_copy(
            bufA, out_hbm.at[b, pl.ds(d0, DG), pl.ds(0, HA)], sem_wA)

        if pend_B is not None:
            pend_B.wait()

        @plsc.parallel_loop(0, (WB - 1) // LANES, unroll=8)
        def _stepB(j):
            iv = plsc.load_gather(idxv,
                                  [iota + (j * LANES + (PAD - 1 + HA))])
            off = pl.multiple_of(j * LANES, LANES)
            for dd in range(DG):
                gv = plsc.load_gather(tvm, [iv + (dd * TABLE)])
                bufB[dd, pl.ds(off, LANES)] = gv

        ivt = plsc.load_gather(idxv, [iota + (PAD - 1 + N)])
        for dd in range(DG):
            gv = plsc.load_gather(tvm, [ivt + (dd * TABLE)])
            plsc.store_scatter(bufB, [jnp.full((LANES,), dd, jnp.int32),
                                      jnp.full((LANES,), WB - 1, jnp.int32)],
                               gv, mask=iota < 1)
        pend_B = pltpu.async_copy(
            bufB, out_hbm.at[b, pl.ds(d0, DG), pl.ds(HA, WB)], sem_wB)

    pend_A.wait()
    pend_B.wait()


@functools.partial(
    pl.kernel,
    mesh=plsc.VectorSubcoreMesh(core_axis_name="c", subcore_axis_name="s"),
    compiler_params=pltpu.CompilerParams(use_tc_tiling_on_sc=True,
                                         needs_layout_passes=False),
    out_type=jax.ShapeDtypeStruct((B, D, N + 1), jnp.float32),
    scratch_types=[
        pltpu.VMEM((PAD + N + LANES,), jnp.int32),
        pltpu.VMEM((PAD + N + LANES,), jnp.int32),
        pltpu.VMEM((DG * TABLE,), jnp.float32),
        pltpu.VMEM((DG, HA), jnp.float32),
        pltpu.VMEM((DG, WB), jnp.float32),
        pltpu.VMEM((D + LANES,), jnp.float32),
        pltpu.SemaphoreType.DMA,
        pltpu.SemaphoreType.DMA,
        pltpu.SemaphoreType.DMA,
        pltpu.SemaphoreType.DMA,
        pltpu.SemaphoreType.DMA,
    ],
)
def _sc_kernel(pos_hbm, op_hbm, t_hbm, out_hbm,
               idxv0, idxv1, tvm, bufA, bufB, opvm,
               sem_tab, sem_idx0, sem_idx1, sem_wA, sem_wB):
    _sc_body(pos_hbm, op_hbm, t_hbm, out_hbm,
             idxv0, idxv1, tvm, bufA, bufB, opvm,
             sem_tab, sem_idx0, sem_idx1, sem_wA, sem_wB)


def kernel(x, y, z, outputPos, posEmbeddingList):
    pos = _pos_kernel(x, y, z)
    tflat = posEmbeddingList.T.reshape(-1)
    op_pad = jnp.pad(outputPos.reshape(-1), (0, LANES))
    out_t = _sc_kernel(pos, op_pad, tflat)
    return out_t.transpose(0, 2, 1)


# speedup vs baseline: 1.2240x; 1.0373x over previous
import functools

import jax
import jax.numpy as jnp
from jax import lax
from jax.experimental import pallas as pl
from jax.experimental.pallas import tpu as pltpu
from jax.experimental.pallas import tpu_sc as plsc

RES = 16
D = 64
B = 32
N = 8192
TABLE = RES ** 3

NC = 2
NS = 16
DG = 8
NB_SC = 16

LANES = 16
PAD = 16
HA = 4096
WB = N + 1 - HA


@functools.partial(
    pl.pallas_call,
    out_shape=jax.ShapeDtypeStruct((B * N,), jnp.int32),
    grid=(B // 8,),
    in_specs=[
        pl.BlockSpec((8, N), lambda q: (q, 0)),
        pl.BlockSpec((8, N), lambda q: (q, 0)),
        pl.BlockSpec((8, N), lambda q: (q, 0)),
    ],
    out_specs=pl.BlockSpec((8 * N,), lambda q: (q,)),
)
def _pos_kernel(x_ref, y_ref, z_ref, o_ref):
    xi = jnp.round(x_ref[...] * 15.0).astype(jnp.int32)
    yi = jnp.round(y_ref[...] * 15.0).astype(jnp.int32)
    zi = jnp.round(z_ref[...] * 15.0).astype(jnp.int32)
    o_ref[...] = (xi * 256 + yi * 16 + zi).reshape(8 * N)


def _sc_body(pos_hbm, op_hbm, t_hbm, out_hbm,
             idxv0, idxv1, tvm, bufA, bufB, opvm,
             sem_tab, sem_idx0, sem_idx1, sem_wA, sem_wB):
    c = lax.axis_index("c")
    s = lax.axis_index("s")
    g = s % DG
    sg = s // DG
    d0 = pl.multiple_of(g * DG, DG)

    iota = lax.iota(jnp.int32, LANES)

    tab_cp = pltpu.async_copy(
        t_hbm.at[pl.ds(pl.multiple_of(g * (DG * TABLE), 128), DG * TABLE)],
        tvm, sem_tab)
    pltpu.sync_copy(op_hbm, opvm)

    ibufs = (idxv0, idxv1)
    isems = (sem_idx0, sem_idx1)
    for ib in ibufs:
        ib[pl.ds(0, PAD)] = jnp.zeros((PAD,), jnp.int32)
        ib[pl.ds(PAD + N, LANES)] = jnp.zeros((LANES,), jnp.int32)

    icps = [pltpu.async_copy(
        pos_hbm.at[pl.ds(pl.multiple_of((c * NB_SC + sg * DG) * N, N), N)],
        idxv0.at[pl.ds(PAD, N)], sem_idx0), None]
    tab_cp.wait()

    pend_A = pend_B = None
    for k in range(DG):
        b = c * NB_SC + sg * DG + k
        idxv = ibufs[k % 2]
        icps[k % 2].wait()
        if k + 1 < DG:
            nxt = pl.multiple_of((c * NB_SC + sg * DG + (k + 1)) * N, N)
            icps[(k + 1) % 2] = pltpu.async_copy(
                pos_hbm.at[pl.ds(nxt, N)],
                ibufs[(k + 1) % 2].at[pl.ds(PAD, N)], isems[(k + 1) % 2])

        if pend_A is not None:
            pend_A.wait()

        @plsc.parallel_loop(0, HA // LANES, unroll=8)
        def _stepA(j):
            iv = plsc.load_gather(idxv, [iota + (j * LANES + (PAD - 1))])
            off = pl.multiple_of(j * LANES, LANES)
            for dd in range(DG):
                gv = plsc.load_gather(tvm, [iv + (dd * TABLE)])
                bufA[dd, pl.ds(off, LANES)] = gv

        opvec = plsc.load_gather(opvm, [jnp.zeros((LANES,), jnp.int32),
                                        iota + d0], mask=iota < DG)
        plsc.store_scatter(bufA, [iota, jnp.zeros((LANES,), jnp.int32)],
                           opvec, mask=iota < DG)
        pend_A = pltpu.async_copy(
            bufA, out_hbm.at[b, pl.ds(d0, DG), pl.ds(0, HA)], sem_wA)

        if pend_B is not None:
            pend_B.wait()

        @plsc.parallel_loop(0, (WB - 1) // LANES, unroll=8)
        def _stepB(j):
            iv = plsc.load_gather(idxv,
                                  [iota + (j * LANES + (PAD - 1 + HA))])
            off = pl.multiple_of(j * LANES, LANES)
            for dd in range(DG):
                gv = plsc.load_gather(tvm, [iv + (dd * TABLE)])
                bufB[dd, pl.ds(off, LANES)] = gv

        ivt = plsc.load_gather(idxv, [iota + (PAD - 1 + N)])
        for dd in range(DG):
            gv = plsc.load_gather(tvm, [ivt + (dd * TABLE)])
            plsc.store_scatter(bufB, [jnp.full((LANES,), dd, jnp.int32),
                                      jnp.full((LANES,), WB - 1, jnp.int32)],
                               gv, mask=iota < 1)
        pend_B = pltpu.async_copy(
            bufB, out_hbm.at[b, pl.ds(d0, DG), pl.ds(HA, WB)], sem_wB)

    pend_A.wait()
    pend_B.wait()


@functools.partial(
    pl.kernel,
    mesh=plsc.VectorSubcoreMesh(core_axis_name="c", subcore_axis_name="s"),
    compiler_params=pltpu.CompilerParams(use_tc_tiling_on_sc=True,
                                         needs_layout_passes=False),
    out_type=jax.ShapeDtypeStruct((B, D, N + 1), jnp.float32),
    scratch_types=[
        pltpu.VMEM((PAD + N + LANES,), jnp.int32),
        pltpu.VMEM((PAD + N + LANES,), jnp.int32),
        pltpu.VMEM((DG * TABLE,), jnp.float32),
        pltpu.VMEM((DG, HA), jnp.float32),
        pltpu.VMEM((DG, WB), jnp.float32),
        pltpu.VMEM((1, D), jnp.float32),
        pltpu.SemaphoreType.DMA,
        pltpu.SemaphoreType.DMA,
        pltpu.SemaphoreType.DMA,
        pltpu.SemaphoreType.DMA,
        pltpu.SemaphoreType.DMA,
    ],
)
def _sc_kernel(pos_hbm, op_hbm, t_hbm, out_hbm,
               idxv0, idxv1, tvm, bufA, bufB, opvm,
               sem_tab, sem_idx0, sem_idx1, sem_wA, sem_wB):
    _sc_body(pos_hbm, op_hbm, t_hbm, out_hbm,
             idxv0, idxv1, tvm, bufA, bufB, opvm,
             sem_tab, sem_idx0, sem_idx1, sem_wA, sem_wB)


def kernel(x, y, z, outputPos, posEmbeddingList):
    pos = _pos_kernel(x, y, z)
    tflat = posEmbeddingList.T.reshape(-1)
    out_t = _sc_kernel(pos, outputPos, tflat)
    return out_t.transpose(0, 2, 1)


# speedup vs baseline: 1.2285x; 1.0037x over previous
import functools

import jax
import jax.numpy as jnp
from jax import lax
from jax.experimental import pallas as pl
from jax.experimental.pallas import tpu as pltpu
from jax.experimental.pallas import tpu_sc as plsc

RES = 16
D = 64
B = 32
N = 8192
TABLE = RES ** 3

NC = 2
NS = 16
DG = 8
NB_SC = 16

LANES = 16
PAD = 16
HA = 4096
WB = N + 1 - HA


@functools.partial(
    pl.pallas_call,
    out_shape=jax.ShapeDtypeStruct((B * N,), jnp.int32),
    grid=(B // 8,),
    in_specs=[
        pl.BlockSpec((8, N), lambda q: (q, 0)),
        pl.BlockSpec((8, N), lambda q: (q, 0)),
        pl.BlockSpec((8, N), lambda q: (q, 0)),
    ],
    out_specs=pl.BlockSpec((8 * N,), lambda q: (q,)),
)
def _pos_kernel(x_ref, y_ref, z_ref, o_ref):
    xi = jnp.round(x_ref[...] * 15.0).astype(jnp.int32)
    yi = jnp.round(y_ref[...] * 15.0).astype(jnp.int32)
    zi = jnp.round(z_ref[...] * 15.0).astype(jnp.int32)
    o_ref[...] = (xi * 256 + yi * 16 + zi).reshape(8 * N)


def _sc_body(pos_hbm, op_hbm, t_hbm, out_hbm,
             idxv0, idxv1, tvm, bufA, bufB, opvm,
             sem_tab, sem_idx0, sem_idx1, sem_wA, sem_wB):
    c = lax.axis_index("c")
    s = lax.axis_index("s")
    g = s % DG
    sg = s // DG
    d0 = pl.multiple_of(g * DG, DG)

    iota = lax.iota(jnp.int32, LANES)

    tab_cp = pltpu.async_copy(
        t_hbm.at[pl.ds(d0, DG)], tvm, sem_tab)
    pltpu.sync_copy(op_hbm, opvm)

    ibufs = (idxv0, idxv1)
    isems = (sem_idx0, sem_idx1)
    for ib in ibufs:
        ib[pl.ds(0, PAD)] = jnp.zeros((PAD,), jnp.int32)
        ib[pl.ds(PAD + N, LANES)] = jnp.zeros((LANES,), jnp.int32)

    icps = [pltpu.async_copy(
        pos_hbm.at[pl.ds(pl.multiple_of((c * NB_SC + sg * DG) * N, N), N)],
        idxv0.at[pl.ds(PAD, N)], sem_idx0), None]
    tab_cp.wait()

    pend_A = pend_B = None
    for k in range(DG):
        b = c * NB_SC + sg * DG + k
        idxv = ibufs[k % 2]
        icps[k % 2].wait()
        if k + 1 < DG:
            nxt = pl.multiple_of((c * NB_SC + sg * DG + (k + 1)) * N, N)
            icps[(k + 1) % 2] = pltpu.async_copy(
                pos_hbm.at[pl.ds(nxt, N)],
                ibufs[(k + 1) % 2].at[pl.ds(PAD, N)], isems[(k + 1) % 2])

        if pend_A is not None:
            pend_A.wait()

        @plsc.parallel_loop(0, HA // LANES, unroll=8)
        def _stepA(j):
            iv = plsc.load_gather(idxv, [iota + (j * LANES + (PAD - 1))])
            off = pl.multiple_of(j * LANES, LANES)
            for dd in range(DG):
                gv = plsc.load_gather(tvm, [jnp.full((LANES,), dd, jnp.int32),
                                            iv])
                bufA[dd, pl.ds(off, LANES)] = gv

        opvec = plsc.load_gather(opvm, [jnp.zeros((LANES,), jnp.int32),
                                        iota + d0], mask=iota < DG)
        plsc.store_scatter(bufA, [iota, jnp.zeros((LANES,), jnp.int32)],
                           opvec, mask=iota < DG)
        pend_A = pltpu.async_copy(
            bufA, out_hbm.at[b, pl.ds(d0, DG), pl.ds(0, HA)], sem_wA)

        if pend_B is not None:
            pend_B.wait()

        @plsc.parallel_loop(0, (WB - 1) // LANES, unroll=8)
        def _stepB(j):
            iv = plsc.load_gather(idxv,
                                  [iota + (j * LANES + (PAD - 1 + HA))])
            off = pl.multiple_of(j * LANES, LANES)
            for dd in range(DG):
                gv = plsc.load_gather(tvm, [jnp.full((LANES,), dd, jnp.int32),
                                            iv])
                bufB[dd, pl.ds(off, LANES)] = gv

        ivt = plsc.load_gather(idxv, [iota + (PAD - 1 + N)])
        for dd in range(DG):
            gv = plsc.load_gather(tvm, [jnp.full((LANES,), dd, jnp.int32),
                                        ivt])
            plsc.store_scatter(bufB, [jnp.full((LANES,), dd, jnp.int32),
                                      jnp.full((LANES,), WB - 1, jnp.int32)],
                               gv, mask=iota < 1)
        pend_B = pltpu.async_copy(
            bufB, out_hbm.at[b, pl.ds(d0, DG), pl.ds(HA, WB)], sem_wB)

    pend_A.wait()
    pend_B.wait()


@functools.partial(
    pl.kernel,
    mesh=plsc.VectorSubcoreMesh(core_axis_name="c", subcore_axis_name="s"),
    compiler_params=pltpu.CompilerParams(use_tc_tiling_on_sc=True,
                                         needs_layout_passes=False),
    out_type=jax.ShapeDtypeStruct((B, D, N + 1), jnp.float32),
    scratch_types=[
        pltpu.VMEM((PAD + N + LANES,), jnp.int32),
        pltpu.VMEM((PAD + N + LANES,), jnp.int32),
        pltpu.VMEM((DG, TABLE), jnp.float32),
        pltpu.VMEM((DG, HA), jnp.float32),
        pltpu.VMEM((DG, WB), jnp.float32),
        pltpu.VMEM((1, D), jnp.float32),
        pltpu.SemaphoreType.DMA,
        pltpu.SemaphoreType.DMA,
        pltpu.SemaphoreType.DMA,
        pltpu.SemaphoreType.DMA,
        pltpu.SemaphoreType.DMA,
    ],
)
def _sc_kernel(pos_hbm, op_hbm, t_hbm, out_hbm,
               idxv0, idxv1, tvm, bufA, bufB, opvm,
               sem_tab, sem_idx0, sem_idx1, sem_wA, sem_wB):
    _sc_body(pos_hbm, op_hbm, t_hbm, out_hbm,
             idxv0, idxv1, tvm, bufA, bufB, opvm,
             sem_tab, sem_idx0, sem_idx1, sem_wA, sem_wB)


def kernel(x, y, z, outputPos, posEmbeddingList):
    pos = _pos_kernel(x, y, z)
    out_t = _sc_kernel(pos, outputPos, posEmbeddingList.T)
    return out_t.transpose(0, 2, 1)
